# baseline (speedup 1.0000x reference)
import functools

import jax
import jax.numpy as jnp
from jax import lax
from jax.experimental import pallas as pl
from jax.experimental.pallas import tpu as pltpu
from jax.experimental.pallas import tpu_sc as plsc

MAXLEN = 2048
D_MODEL = 1024

NC = 2
NS = 16
NW = NC * NS
SPW = MAXLEN // NW
CH = 8
CHW = CH * D_MODEL
UNROLL = 4


def _sc_kernel_body(B, x_hbm, pos_hbm, out_hbm,
                    xbuf0, xbuf1, pbuf0, pbuf1,
                    lsem0, lsem1, ssem0, ssem1):
    xbufs = (xbuf0, xbuf1)
    pbufs = (pbuf0, pbuf1)
    lsems = (lsem0, lsem1)
    ssems = (ssem0, ssem1)

    wid = lax.axis_index("s") * NC + lax.axis_index("c")
    s_base = wid * SPW
    n_chunk = SPW // CH

    def start_loads(c):
        p = c % 2
        s0 = (s_base + c * CH) * D_MODEL
        h = [pltpu.async_copy(pos_hbm.at[pl.ds(s0, CHW)], pbufs[p], lsems[p])]
        for b in range(B):
            h.append(pltpu.async_copy(
                x_hbm.at[pl.ds(b * MAXLEN * D_MODEL + s0, CHW)],
                xbufs[p].at[pl.ds(b * CHW, CHW)], lsems[p]))
        return h

    loads = {0: start_loads(0)}
    stores = {}
    for c in range(n_chunk):
        p = c % 2
        for h in loads.pop(c):
            h.wait()
        if c + 1 < n_chunk:
            if c >= 1:
                for h in stores.pop(c - 1):
                    h.wait()
            loads[c + 1] = start_loads(c + 1)

        xb, pb = xbufs[p], pbufs[p]

        def body(i, _):
            for u in range(UNROLL):
                off = (i * UNROLL + u) * 16
                ps = pb[pl.ds(off, 16)]
                for b in range(B):
                    xo = b * CHW + off
                    xb[pl.ds(xo, 16)] = xb[pl.ds(xo, 16)] + ps
            return 0

        lax.fori_loop(0, CHW // (16 * UNROLL), body, 0, unroll=False)

        s0 = (s_base + c * CH) * D_MODEL
        stores[c] = [pltpu.async_copy(
            xbufs[p].at[pl.ds(b * CHW, CHW)],
            out_hbm.at[pl.ds(b * MAXLEN * D_MODEL + s0, CHW)], ssems[p])
            for b in range(B)]
    for hs in stores.values():
        for h in hs:
            h.wait()


def _make_sc_call(B):
    mesh = plsc.VectorSubcoreMesh(core_axis_name="c", subcore_axis_name="s")
    return pl.kernel(
        functools.partial(_sc_kernel_body, B),
        mesh=mesh,
        out_type=jax.ShapeDtypeStruct((B * MAXLEN * D_MODEL,), jnp.float32),
        scratch_types=[
            pltpu.VMEM((4 * CHW,), jnp.float32),
            pltpu.VMEM((4 * CHW,), jnp.float32),
            pltpu.VMEM((CHW,), jnp.float32),
            pltpu.VMEM((CHW,), jnp.float32),
            pltpu.SemaphoreType.DMA,
            pltpu.SemaphoreType.DMA,
            pltpu.SemaphoreType.DMA,
            pltpu.SemaphoreType.DMA,
        ],
    )


def kernel(x, pos_table):
    B, S, D = x.shape
    xf = jnp.reshape(x, (B * S * D,))
    pf = jnp.reshape(pos_table, (S * D,))
    out = _make_sc_call(B)(xf, pf)
    return jnp.reshape(out, (B, S, D))


# speedup vs baseline: 1.0326x; 1.0326x over previous
import functools

import jax
import jax.numpy as jnp
from jax import lax
from jax.experimental import pallas as pl
from jax.experimental.pallas import tpu as pltpu
from jax.experimental.pallas import tpu_sc as plsc

MAXLEN = 2048
D_MODEL = 1024

NC = 2
NS = 16
NW = NC * NS
SPW = MAXLEN // NW
CH = 8
CHW = CH * D_MODEL
UNROLL = 4


def _sc_kernel_body(B, x_hbm, pos_hbm, out_hbm,
                    xbuf0, xbuf1, pbuf0, pbuf1,
                    lsem0, lsem1, ssem0, ssem1):
    xbufs = (xbuf0, xbuf1)
    pbufs = (pbuf0, pbuf1)
    lsems = (lsem0, lsem1)
    ssems = (ssem0, ssem1)

    wid = lax.axis_index("s") * NC + lax.axis_index("c")
    s_base = wid * SPW
    n_chunk = SPW // CH

    def start_loads(c):
        p = c % 2
        s0 = (s_base + c * CH) * D_MODEL
        h = [pltpu.async_copy(pos_hbm.at[pl.ds(s0, CHW)], pbufs[p], lsems[p])]
        for b in range(B):
            h.append(pltpu.async_copy(
                x_hbm.at[pl.ds(b * MAXLEN * D_MODEL + s0, CHW)],
                xbufs[p].at[pl.ds(b * CHW, CHW)], lsems[p]))
        return h

    loads = {0: start_loads(0)}
    stores = {}
    for c in range(n_chunk):
        p = c % 2
        for h in loads.pop(c):
            h.wait()
        if c + 1 < n_chunk:
            if c >= 1:
                for h in stores.pop(c - 1):
                    h.wait()
            loads[c + 1] = start_loads(c + 1)

        xb, pb = xbufs[p], pbufs[p]

        @plsc.parallel_loop(0, CHW // 16, step=1, unroll=UNROLL)
        def _body(i):
            off = i * 16
            ps = pb[pl.ds(off, 16)]
            for b in range(B):
                xo = b * CHW + off
                xb[pl.ds(xo, 16)] = xb[pl.ds(xo, 16)] + ps

        s0 = (s_base + c * CH) * D_MODEL
        stores[c] = [pltpu.async_copy(
            xbufs[p].at[pl.ds(b * CHW, CHW)],
            out_hbm.at[pl.ds(b * MAXLEN * D_MODEL + s0, CHW)], ssems[p])
            for b in range(B)]
    for hs in stores.values():
        for h in hs:
            h.wait()


def _make_sc_call(B):
    mesh = plsc.VectorSubcoreMesh(core_axis_name="c", subcore_axis_name="s")
    return pl.kernel(
        functools.partial(_sc_kernel_body, B),
        mesh=mesh,
        out_type=jax.ShapeDtypeStruct((B * MAXLEN * D_MODEL,), jnp.float32),
        scratch_types=[
            pltpu.VMEM((4 * CHW,), jnp.float32),
            pltpu.VMEM((4 * CHW,), jnp.float32),
            pltpu.VMEM((CHW,), jnp.float32),
            pltpu.VMEM((CHW,), jnp.float32),
            pltpu.SemaphoreType.DMA,
            pltpu.SemaphoreType.DMA,
            pltpu.SemaphoreType.DMA,
            pltpu.SemaphoreType.DMA,
        ],
    )


def kernel(x, pos_table):
    B, S, D = x.shape
    xf = jnp.reshape(x, (B * S * D,))
    pf = jnp.reshape(pos_table, (S * D,))
    out = _make_sc_call(B)(xf, pf)
    return jnp.reshape(out, (B, S, D))
